# Initial kernel scaffold; baseline (speedup 1.0000x reference)
import functools

import jax
import jax.numpy as jnp
from jax import lax
from jax.experimental import pallas as pl
from jax.experimental.pallas import tpu as pltpu
from jax.experimental.pallas import tpu_sc as plsc

N = 10000
F = 128
E = 320000
MAXN = 1.0 - 1e-5

NC = 2
NS = 16
NW = NC * NS
EPW = E // NW
CH = 125
NCHUNK = EPW // CH
STR = N // NS
DEGW = 16

BLK = 2000


def _norm(p):
    return jnp.sqrt(jnp.sum(p * p, axis=-1, keepdims=True))


def _project(p):
    n = _norm(p)
    return jnp.where(n > MAXN, p / jnp.maximum(n, 1e-10) * MAXN, p)


def _expmap0(u):
    n = jnp.maximum(_norm(u), 1e-10)
    return _project(jnp.tanh(n) * u / n)


def _logmap0(p):
    p = _project(p)
    n = jnp.maximum(_norm(p), 1e-10)
    nc = jnp.minimum(n, MAXN)
    atanh = 0.5 * jnp.log((1.0 + nc) / (1.0 - nc))
    return atanh * p / n


def _tcA_body(x_ref, W_ref, b_ref, o_ref):
    x = x_ref[...]
    x0 = x[:, 0:1]
    xs = jnp.concatenate([x[:, 1:], jnp.zeros_like(x0)], axis=1)
    p = xs / (1.0 + x0)
    for i in range(3):
        u = jnp.dot(_logmap0(p), W_ref[i],
                    preferred_element_type=jnp.float32) + b_ref[i]
        p = _expmap0(jnp.maximum(u, 0.0))
    o_ref[...] = jnp.dot(_logmap0(p), W_ref[3],
                         preferred_element_type=jnp.float32) + b_ref[3]


def _mean_relu_exp(agg_ref, deg_ref):
    s = agg_ref[0] + agg_ref[1]
    d = deg_ref[0] + deg_ref[1]
    deg = jnp.sum(d, axis=-1, keepdims=True) * (1.0 / DEGW)
    u = s / jnp.maximum(deg, 1.0)
    return _expmap0(jnp.maximum(u, 0.0))


def _tcB_body(agg_ref, deg_ref, W_ref, b_ref, o_ref):
    p = _mean_relu_exp(agg_ref, deg_ref)
    o_ref[...] = jnp.dot(_logmap0(p), W_ref[...],
                         preferred_element_type=jnp.float32) + b_ref[...]


def _tcC_body(agg_ref, deg_ref, W_ref, b_ref, o_ref):
    p = _mean_relu_exp(agg_ref, deg_ref)
    u = jnp.dot(_logmap0(p), W_ref[...],
                preferred_element_type=jnp.float32) + b_ref[...]
    o_ref[...] = _expmap0(u)


def _tcA(x, Wst, bst):
    return pl.pallas_call(
        _tcA_body,
        grid=(N // BLK,),
        in_specs=[
            pl.BlockSpec((BLK, F), lambda i: (i, 0)),
            pl.BlockSpec((4, F, F), lambda i: (0, 0, 0)),
            pl.BlockSpec((4, 1, F), lambda i: (0, 0, 0)),
        ],
        out_specs=pl.BlockSpec((BLK, F), lambda i: (i, 0)),
        out_shape=jax.ShapeDtypeStruct((N, F), jnp.float32),
    )(x, Wst, bst)


def _tc_post(body, agg, deg, W, b):
    return pl.pallas_call(
        body,
        grid=(N // BLK,),
        in_specs=[
            pl.BlockSpec((NC, BLK, F), lambda i: (0, i, 0)),
            pl.BlockSpec((NC, BLK, DEGW), lambda i: (0, i, 0)),
            pl.BlockSpec((F, F), lambda i: (0, 0)),
            pl.BlockSpec((1, F), lambda i: (0, 0)),
        ],
        out_specs=pl.BlockSpec((BLK, F), lambda i: (i, 0)),
        out_shape=jax.ShapeDtypeStruct((N, F), jnp.float32),
    )(agg, deg, W, b)


_sc_mesh = plsc.VectorSubcoreMesh(core_axis_name="c", subcore_axis_name="s")


@functools.partial(
    pl.kernel,
    out_type=(
        jax.ShapeDtypeStruct((NC, N, F), jnp.float32),
        jax.ShapeDtypeStruct((NC, N, DEGW), jnp.float32),
    ),
    mesh=_sc_mesh,
    scratch_types=[
        pltpu.VMEM((NCHUNK, CH), jnp.int32),
        pltpu.VMEM((NCHUNK, CH), jnp.int32),
        pltpu.VMEM((CH, F), jnp.float32),
        pltpu.VMEM((CH, DEGW), jnp.float32),
        pltpu.VMEM_SHARED((N, F), jnp.float32),
        pltpu.VMEM_SHARED((N, DEGW), jnp.float32),
        pltpu.SemaphoreType.DMA,
    ],
)
def _sc_agg(u_hbm, src_hbm, dst_hbm, zf_hbm, zd_hbm, ones_hbm,
            out_hbm, deg_hbm,
            src_v, dst_v, rows_v, ones_v, acc_sh, dacc_sh, sem):
    cid = lax.axis_index("c")
    sid = lax.axis_index("s")
    wid = sid * NC + cid

    pltpu.sync_copy(src_hbm.at[wid], src_v)
    pltpu.sync_copy(dst_hbm.at[wid], dst_v)
    pltpu.sync_copy(ones_hbm, ones_v)
    pltpu.sync_copy(zf_hbm, acc_sh.at[pl.ds(sid * STR, STR)])
    pltpu.sync_copy(zd_hbm, dacc_sh.at[pl.ds(sid * STR, STR)])
    plsc.subcore_barrier()

    def body(j, carry):
        pltpu.async_copy(u_hbm.at[src_v.at[j]], rows_v, sem).wait()
        pltpu.sync_copy(rows_v, acc_sh.at[dst_v.at[j]], add=True)
        pltpu.sync_copy(ones_v, dacc_sh.at[dst_v.at[j]], add=True)
        return carry

    lax.fori_loop(0, NCHUNK, body, 0)
    plsc.subcore_barrier()

    row = pl.ds(sid * STR, STR)
    pltpu.sync_copy(acc_sh.at[row], out_hbm.at[cid, row])
    pltpu.sync_copy(dacc_sh.at[row], deg_hbm.at[cid, row])


def kernel(x, edge_index, W1, b1, W2, b2, W3, b3, W4, b4, W5, b5, W6, b6):
    W1p = jnp.concatenate([W1, jnp.zeros((1, F), W1.dtype)], axis=0)
    Wst = jnp.stack([W1p, W2, W3, W4])
    bst = jnp.stack([b1, b2, b3, b4])[:, None, :]

    src = edge_index[0].reshape(NW, NCHUNK, CH)
    dst = edge_index[1].reshape(NW, NCHUNK, CH)
    zf = jnp.zeros((STR, F), jnp.float32)
    zd = jnp.zeros((STR, DEGW), jnp.float32)
    ones = jnp.ones((CH, DEGW), jnp.float32)

    u4 = _tcA(x, Wst, bst)
    agg1, deg1 = _sc_agg(u4, src, dst, zf, zd, ones)
    u5 = _tc_post(_tcB_body, agg1, deg1, W5, b5[None])
    agg2, deg2 = _sc_agg(u5, src, dst, zf, zd, ones)
    return _tc_post(_tcC_body, agg2, deg2, W6, b6[None])


# speedup vs baseline: 5.8282x; 5.8282x over previous
import functools

import jax
import jax.numpy as jnp
from jax import lax
from jax.experimental import pallas as pl
from jax.experimental.pallas import tpu as pltpu
from jax.experimental.pallas import tpu_sc as plsc

N = 10000
F = 128
E = 320000
MAXN = 1.0 - 1e-5

NC = 2
NS = 16
FH = F // NC
EPT = E // NS
CH = 125
NCHUNK = EPT // CH
NP = 10240
STR = NP // NS
DEGW = 8

BLK = 2000


def _norm(p):
    return jnp.sqrt(jnp.sum(p * p, axis=-1, keepdims=True))


def _project(p):
    n = _norm(p)
    return jnp.where(n > MAXN, p / jnp.maximum(n, 1e-10) * MAXN, p)


def _expmap0(u):
    n = jnp.maximum(_norm(u), 1e-10)
    return _project(jnp.tanh(n) * u / n)


def _logmap0(p):
    p = _project(p)
    n = jnp.maximum(_norm(p), 1e-10)
    nc = jnp.minimum(n, MAXN)
    atanh = 0.5 * jnp.log((1.0 + nc) / (1.0 - nc))
    return atanh * p / n


def _split_store(o_ref, u):
    o_ref[0] = u[:, :FH]
    o_ref[1] = u[:, FH:]


def _tcA_body(x_ref, W_ref, b_ref, o_ref):
    x = x_ref[...]
    x0 = x[:, 0:1]
    xs = jnp.concatenate([x[:, 1:], jnp.zeros_like(x0)], axis=1)
    p = xs / (1.0 + x0)
    for i in range(3):
        u = jnp.dot(_logmap0(p), W_ref[i],
                    preferred_element_type=jnp.float32) + b_ref[i]
        p = _expmap0(jnp.maximum(u, 0.0))
    _split_store(o_ref, jnp.dot(_logmap0(p), W_ref[3],
                                preferred_element_type=jnp.float32) + b_ref[3])


def _mean_relu_exp(agg_ref, deg_ref):
    s = jnp.concatenate([agg_ref[0], agg_ref[1]], axis=-1)
    deg = jnp.sum(deg_ref[...], axis=-1, keepdims=True) * (1.0 / DEGW)
    u = s / jnp.maximum(deg, 1.0)
    return _expmap0(jnp.maximum(u, 0.0))


def _tcB_body(agg_ref, deg_ref, W_ref, b_ref, o_ref):
    p = _mean_relu_exp(agg_ref, deg_ref)
    _split_store(o_ref, jnp.dot(_logmap0(p), W_ref[...],
                                preferred_element_type=jnp.float32) + b_ref[...])


def _tcC_body(agg_ref, deg_ref, W_ref, b_ref, o_ref):
    p = _mean_relu_exp(agg_ref, deg_ref)
    u = jnp.dot(_logmap0(p), W_ref[...],
                preferred_element_type=jnp.float32) + b_ref[...]
    o_ref[...] = _expmap0(u)


def _tcA(x, Wst, bst):
    return pl.pallas_call(
        _tcA_body,
        grid=(N // BLK,),
        in_specs=[
            pl.BlockSpec((BLK, F), lambda i: (i, 0)),
            pl.BlockSpec((4, F, F), lambda i: (0, 0, 0)),
            pl.BlockSpec((4, 1, F), lambda i: (0, 0, 0)),
        ],
        out_specs=pl.BlockSpec((NC, BLK, FH), lambda i: (0, i, 0)),
        out_shape=jax.ShapeDtypeStruct((NC, N, FH), jnp.float32),
    )(x, Wst, bst)


def _tc_post(body, agg, deg, W, b, final):
    out_specs = (pl.BlockSpec((BLK, F), lambda i: (i, 0)) if final
                 else pl.BlockSpec((NC, BLK, FH), lambda i: (0, i, 0)))
    out_shape = (jax.ShapeDtypeStruct((N, F), jnp.float32) if final
                 else jax.ShapeDtypeStruct((NC, N, FH), jnp.float32))
    return pl.pallas_call(
        body,
        grid=(N // BLK,),
        in_specs=[
            pl.BlockSpec((NC, BLK, FH), lambda i: (0, i, 0)),
            pl.BlockSpec((BLK, DEGW), lambda i: (i, 0)),
            pl.BlockSpec((F, F), lambda i: (0, 0)),
            pl.BlockSpec((1, F), lambda i: (0, 0)),
        ],
        out_specs=out_specs,
        out_shape=out_shape,
    )(agg, deg, W, b)


@functools.cache
def _make_sc_agg():
    mesh = plsc.VectorSubcoreMesh(
        core_axis_name="c", subcore_axis_name="s",
        num_cores=NC, num_subcores=NS)

    @functools.partial(
        pl.kernel,
        out_type=(
            jax.ShapeDtypeStruct((NC, NP, FH), jnp.float32),
            jax.ShapeDtypeStruct((NP, DEGW), jnp.float32),
        ),
        mesh=mesh,
        compiler_params=pltpu.CompilerParams(use_tc_tiling_on_sc=False),
        scratch_types=[
            pltpu.VMEM((NCHUNK, CH), jnp.int32),
            pltpu.VMEM((NCHUNK, CH), jnp.int32),
            pltpu.VMEM((CH, FH), jnp.float32),
            pltpu.VMEM((CH, DEGW), jnp.float32),
            pltpu.VMEM_SHARED((NP, FH), jnp.float32),
            pltpu.VMEM_SHARED((NP, DEGW), jnp.float32),
            pltpu.SemaphoreType.DMA,
        ],
    )
    def _sc_agg(u_hbm, src_hbm, dst_hbm, zf_hbm, zd_hbm, ones_hbm,
                out_hbm, deg_hbm,
                src_v, dst_v, rows_v, ones_v, acc_sh, dacc_sh, sem):
        cid = lax.axis_index("c")
        sid = lax.axis_index("s")
        on0 = cid == 0

        pltpu.sync_copy(src_hbm.at[sid], src_v)
        pltpu.sync_copy(dst_hbm.at[sid], dst_v)
        pltpu.sync_copy(ones_hbm, ones_v)
        pltpu.sync_copy(zf_hbm, acc_sh.at[pl.ds(sid * STR, STR)])

        @pl.when(on0)
        def _():
            pltpu.sync_copy(zd_hbm, dacc_sh.at[pl.ds(sid * STR, STR)])

        plsc.subcore_barrier()

        def body(j, carry):
            pltpu.async_copy(u_hbm.at[cid].at[src_v.at[j]], rows_v, sem).wait()
            pltpu.sync_copy(rows_v, acc_sh.at[dst_v.at[j]], add=True)

            @pl.when(on0)
            def _():
                pltpu.sync_copy(ones_v, dacc_sh.at[dst_v.at[j]], add=True)

            return carry

        lax.fori_loop(0, NCHUNK, body, 0)
        plsc.subcore_barrier()

        row = pl.ds(sid * STR, STR)
        pltpu.sync_copy(acc_sh.at[row], out_hbm.at[cid, row])

        @pl.when(on0)
        def _():
            pltpu.sync_copy(dacc_sh.at[row], deg_hbm.at[row])

    return _sc_agg


def kernel(x, edge_index, W1, b1, W2, b2, W3, b3, W4, b4, W5, b5, W6, b6):
    W1p = jnp.concatenate([W1, jnp.zeros((1, F), W1.dtype)], axis=0)
    Wst = jnp.stack([W1p, W2, W3, W4])
    bst = jnp.stack([b1, b2, b3, b4])[:, None, :]

    src = edge_index[0].reshape(NS, NCHUNK, CH)
    dst = edge_index[1].reshape(NS, NCHUNK, CH)
    zf = jnp.zeros((STR, FH), jnp.float32)
    zd = jnp.zeros((STR, DEGW), jnp.float32)
    ones = jnp.ones((CH, DEGW), jnp.float32)

    sc_agg = _make_sc_agg()
    u4 = _tcA(x, Wst, bst)
    agg1, deg1 = sc_agg(u4, src, dst, zf, zd, ones)
    u5 = _tc_post(_tcB_body, agg1, deg1, W5, b5[None], final=False)
    agg2, deg2 = sc_agg(u5, src, dst, zf, zd, ones)
    return _tc_post(_tcC_body, agg2, deg2, W6, b6[None], final=True)


# speedup vs baseline: 8.9291x; 1.5320x over previous
import functools

import jax
import jax.numpy as jnp
from jax import lax
from jax.experimental import pallas as pl
from jax.experimental.pallas import tpu as pltpu
from jax.experimental.pallas import tpu_sc as plsc

N = 10000
F = 128
E = 320000
MAXN = 1.0 - 1e-5

NC = 2
NS = 16
FH = F // NC
EPT = E // NS
CH = 125
NCHUNK = EPT // CH
NP = 10240
STR = NP // NS
DEGW = 8

BLK = 2000


def _norm(p):
    return jnp.sqrt(jnp.sum(p * p, axis=-1, keepdims=True))


def _project(p):
    n = _norm(p)
    return jnp.where(n > MAXN, p / jnp.maximum(n, 1e-10) * MAXN, p)


def _expmap0(u):
    n = jnp.maximum(_norm(u), 1e-10)
    return _project(jnp.tanh(n) * u / n)


def _logmap0(p):
    p = _project(p)
    n = jnp.maximum(_norm(p), 1e-10)
    nc = jnp.minimum(n, MAXN)
    atanh = 0.5 * jnp.log((1.0 + nc) / (1.0 - nc))
    return atanh * p / n


def _split_store(o_ref, u):
    o_ref[0] = u[:, :FH]
    o_ref[1] = u[:, FH:]


def _tcA_body(x_ref, W_ref, b_ref, o_ref):
    x = x_ref[...]
    x0 = x[:, 0:1]
    xs = jnp.concatenate([x[:, 1:], jnp.zeros_like(x0)], axis=1)
    p = xs / (1.0 + x0)
    for i in range(3):
        u = jnp.dot(_logmap0(p), W_ref[i],
                    preferred_element_type=jnp.float32) + b_ref[i]
        p = _expmap0(jnp.maximum(u, 0.0))
    _split_store(o_ref, jnp.dot(_logmap0(p), W_ref[3],
                                preferred_element_type=jnp.float32) + b_ref[3])


def _mean_relu_exp(agg_ref, deg_ref):
    s = jnp.concatenate([agg_ref[0], agg_ref[1]], axis=-1)
    deg = jnp.sum(deg_ref[...], axis=-1, keepdims=True) * (1.0 / DEGW)
    u = s / jnp.maximum(deg, 1.0)
    return _expmap0(jnp.maximum(u, 0.0))


def _tcB_body(agg_ref, deg_ref, W_ref, b_ref, o_ref):
    p = _mean_relu_exp(agg_ref, deg_ref)
    _split_store(o_ref, jnp.dot(_logmap0(p), W_ref[...],
                                preferred_element_type=jnp.float32) + b_ref[...])


def _tcC_body(agg_ref, deg_ref, W_ref, b_ref, o_ref):
    p = _mean_relu_exp(agg_ref, deg_ref)
    u = jnp.dot(_logmap0(p), W_ref[...],
                preferred_element_type=jnp.float32) + b_ref[...]
    o_ref[...] = _expmap0(u)


def _tcA(x, Wst, bst):
    return pl.pallas_call(
        _tcA_body,
        grid=(N // BLK,),
        in_specs=[
            pl.BlockSpec((BLK, F), lambda i: (i, 0)),
            pl.BlockSpec((4, F, F), lambda i: (0, 0, 0)),
            pl.BlockSpec((4, 1, F), lambda i: (0, 0, 0)),
        ],
        out_specs=pl.BlockSpec((NC, BLK, FH), lambda i: (0, i, 0)),
        out_shape=jax.ShapeDtypeStruct((NC, N, FH), jnp.float32),
    )(x, Wst, bst)


def _tc_post(body, agg, deg, W, b, final):
    out_specs = (pl.BlockSpec((BLK, F), lambda i: (i, 0)) if final
                 else pl.BlockSpec((NC, BLK, FH), lambda i: (0, i, 0)))
    out_shape = (jax.ShapeDtypeStruct((N, F), jnp.float32) if final
                 else jax.ShapeDtypeStruct((NC, N, FH), jnp.float32))
    return pl.pallas_call(
        body,
        grid=(N // BLK,),
        in_specs=[
            pl.BlockSpec((NC, BLK, FH), lambda i: (0, i, 0)),
            pl.BlockSpec((BLK, DEGW), lambda i: (i, 0)),
            pl.BlockSpec((F, F), lambda i: (0, 0)),
            pl.BlockSpec((1, F), lambda i: (0, 0)),
        ],
        out_specs=out_specs,
        out_shape=out_shape,
    )(agg, deg, W, b)


NBUF = 2


@functools.cache
def _make_sc_agg(with_deg):
    mesh = plsc.VectorSubcoreMesh(
        core_axis_name="c", subcore_axis_name="s",
        num_cores=NC, num_subcores=NS)

    @functools.partial(
        pl.kernel,
        out_type=(
            jax.ShapeDtypeStruct((NC, NP, FH), jnp.float32),
            jax.ShapeDtypeStruct((NP, DEGW), jnp.float32),
        ),
        mesh=mesh,
        compiler_params=pltpu.CompilerParams(use_tc_tiling_on_sc=False),
        scratch_types=[
            pltpu.VMEM((NCHUNK, CH), jnp.int32),
            pltpu.VMEM((NCHUNK, CH), jnp.int32),
            [pltpu.VMEM((CH, FH), jnp.float32) for _ in range(NBUF)],
            pltpu.VMEM((CH, DEGW), jnp.float32),
            pltpu.VMEM_SHARED((NP, FH), jnp.float32),
            pltpu.VMEM_SHARED((NP, DEGW), jnp.float32),
            [pltpu.SemaphoreType.DMA for _ in range(NBUF)],
        ],
    )
    def _sc_agg(u_hbm, src_hbm, dst_hbm, zf_hbm, zd_hbm, ones_hbm,
                out_hbm, deg_hbm,
                src_v, dst_v, rows, ones_v, acc_sh, dacc_sh, sems):
        cid = lax.axis_index("c")
        sid = lax.axis_index("s")
        on0 = cid == 0
        uh = u_hbm.at[cid]

        pltpu.sync_copy(src_hbm.at[sid], src_v)
        pltpu.sync_copy(dst_hbm.at[sid], dst_v)
        if with_deg:
            pltpu.sync_copy(ones_hbm, ones_v)
        pltpu.sync_copy(zf_hbm, acc_sh.at[pl.ds(sid * STR, STR)])
        if with_deg:
            pltpu.sync_copy(zd_hbm, dacc_sh.at[pl.ds(sid * STR, STR)])
        plsc.subcore_barrier()

        def gather(j, b):
            return pltpu.make_async_copy(uh.at[src_v.at[j]], rows[b], sems[b])

        def scat(j, b):
            pltpu.sync_copy(rows[b], acc_sh.at[dst_v.at[j]], add=True)
            if with_deg:

                @pl.when(on0)
                def _():
                    pltpu.sync_copy(ones_v, dacc_sh.at[dst_v.at[j]], add=True)

        for b in range(NBUF):
            gather(b, b).start()

        def body(i, carry):
            j = NBUF * i
            for b in range(NBUF):
                gather(j + b, b).wait()
                scat(j + b, b)
                gather(j + b + NBUF, b).start()
            return carry

        lax.fori_loop(0, NCHUNK // NBUF - 1, body, 0)
        j_last = NCHUNK - NBUF
        for b in range(NBUF):
            gather(j_last + b, b).wait()
            scat(j_last + b, b)

        plsc.subcore_barrier()
        row = pl.ds(sid * STR, STR)
        pltpu.sync_copy(acc_sh.at[row], out_hbm.at[cid, row])
        if with_deg:
            pltpu.sync_copy(dacc_sh.at[row], deg_hbm.at[row])

    return _sc_agg


def kernel(x, edge_index, W1, b1, W2, b2, W3, b3, W4, b4, W5, b5, W6, b6):
    W1p = jnp.concatenate([W1, jnp.zeros((1, F), W1.dtype)], axis=0)
    Wst = jnp.stack([W1p, W2, W3, W4])
    bst = jnp.stack([b1, b2, b3, b4])[:, None, :]

    src = edge_index[0].reshape(NS, NCHUNK, CH)
    dst = edge_index[1].reshape(NS, NCHUNK, CH)
    zf = jnp.zeros((STR, FH), jnp.float32)
    zd = jnp.zeros((STR, DEGW), jnp.float32)
    ones = jnp.ones((CH, DEGW), jnp.float32)

    u4 = _tcA(x, Wst, bst)
    agg1, deg1 = _make_sc_agg(True)(u4, src, dst, zf, zd, ones)
    u5 = _tc_post(_tcB_body, agg1, deg1, W5, b5[None], final=False)
    agg2, _ = _make_sc_agg(False)(u5, src, dst, zf, zd, ones)
    return _tc_post(_tcC_body, agg2, deg1, W6, b6[None], final=True)


# speedup vs baseline: 11.8188x; 1.3236x over previous
import functools
import math

import jax
import jax.numpy as jnp
from jax import lax
from jax.experimental import pallas as pl
from jax.experimental.pallas import tpu as pltpu
from jax.experimental.pallas import tpu_sc as plsc

N = 10000
F = 128
E = 320000
MAXN = 1.0 - 1e-5

NC = 2
NS = 16
FH = F // NC
EPT = E // NS
CH = 125
NCHUNK = EPT // CH
NP = 10240
STR = NP // NS
DEGW = 8

BLK = 2000


def _norm(p):
    return jnp.sqrt(jnp.sum(p * p, axis=-1, keepdims=True))


def _project(p):
    n = _norm(p)
    return jnp.where(n > MAXN, p / jnp.maximum(n, 1e-10) * MAXN, p)


def _expmap0(u):
    n = jnp.maximum(_norm(u), 1e-10)
    return _project(jnp.tanh(n) * u / n)


def _logmap0(p):
    p = _project(p)
    n = jnp.maximum(_norm(p), 1e-10)
    nc = jnp.minimum(n, MAXN)
    atanh = 0.5 * jnp.log((1.0 + nc) / (1.0 - nc))
    return atanh * p / n


TCUT = math.atanh(MAXN)


def _roundtrip(v):
    n = jnp.maximum(_norm(v), 1e-10)
    return jnp.where(n >= TCUT, TCUT / n, 1.0) * v


def _split_store(o_ref, u):
    o_ref[0] = u[:, :FH]
    o_ref[1] = u[:, FH:]


def _tcA_body(x_ref, W_ref, b_ref, o_ref):
    x = x_ref[...]
    x0 = x[:, 0:1]
    xs = jnp.concatenate([x[:, 1:], jnp.zeros_like(x0)], axis=1)
    p = xs / (1.0 + x0)
    t = _logmap0(p)
    for i in range(3):
        u = jnp.dot(t, W_ref[i],
                    preferred_element_type=jnp.float32) + b_ref[i]
        t = _roundtrip(jnp.maximum(u, 0.0))
    _split_store(o_ref, jnp.dot(t, W_ref[3],
                                preferred_element_type=jnp.float32) + b_ref[3])


def _mean_relu_tangent(agg_ref, deg_ref):
    s = jnp.concatenate([agg_ref[0], agg_ref[1]], axis=-1)
    deg = jnp.sum(deg_ref[...], axis=-1, keepdims=True) * (1.0 / DEGW)
    u = s / jnp.maximum(deg, 1.0)
    return _roundtrip(jnp.maximum(u, 0.0))


def _tcB_body(agg_ref, deg_ref, W_ref, b_ref, o_ref):
    t = _mean_relu_tangent(agg_ref, deg_ref)
    _split_store(o_ref, jnp.dot(t, W_ref[...],
                                preferred_element_type=jnp.float32) + b_ref[...])


def _tcC_body(agg_ref, deg_ref, W_ref, b_ref, o_ref):
    t = _mean_relu_tangent(agg_ref, deg_ref)
    u = jnp.dot(t, W_ref[...],
                preferred_element_type=jnp.float32) + b_ref[...]
    o_ref[...] = _expmap0(u)


def _tcA(x, Wst, bst):
    return pl.pallas_call(
        _tcA_body,
        grid=(N // BLK,),
        in_specs=[
            pl.BlockSpec((BLK, F), lambda i: (i, 0)),
            pl.BlockSpec((4, F, F), lambda i: (0, 0, 0)),
            pl.BlockSpec((4, 1, F), lambda i: (0, 0, 0)),
        ],
        out_specs=pl.BlockSpec((NC, BLK, FH), lambda i: (0, i, 0)),
        out_shape=jax.ShapeDtypeStruct((NC, N, FH), jnp.float32),
    )(x, Wst, bst)


def _tc_post(body, agg, deg, W, b, final):
    out_specs = (pl.BlockSpec((BLK, F), lambda i: (i, 0)) if final
                 else pl.BlockSpec((NC, BLK, FH), lambda i: (0, i, 0)))
    out_shape = (jax.ShapeDtypeStruct((N, F), jnp.float32) if final
                 else jax.ShapeDtypeStruct((NC, N, FH), jnp.float32))
    return pl.pallas_call(
        body,
        grid=(N // BLK,),
        in_specs=[
            pl.BlockSpec((NC, BLK, FH), lambda i: (0, i, 0)),
            pl.BlockSpec((BLK, DEGW), lambda i: (i, 0)),
            pl.BlockSpec((F, F), lambda i: (0, 0)),
            pl.BlockSpec((1, F), lambda i: (0, 0)),
        ],
        out_specs=out_specs,
        out_shape=out_shape,
    )(agg, deg, W, b)


NBUF = 4


@functools.cache
def _make_sc_agg(with_deg):
    mesh = plsc.VectorSubcoreMesh(
        core_axis_name="c", subcore_axis_name="s",
        num_cores=NC, num_subcores=NS)

    @functools.partial(
        pl.kernel,
        out_type=(
            jax.ShapeDtypeStruct((NC, NP, FH), jnp.float32),
            jax.ShapeDtypeStruct((NP, DEGW), jnp.float32),
        ),
        mesh=mesh,
        compiler_params=pltpu.CompilerParams(use_tc_tiling_on_sc=False),
        scratch_types=[
            pltpu.VMEM((NCHUNK, CH), jnp.int32),
            pltpu.VMEM((NCHUNK, CH), jnp.int32),
            [pltpu.VMEM((CH, FH), jnp.float32) for _ in range(NBUF)],
            pltpu.VMEM((CH, DEGW), jnp.float32),
            pltpu.VMEM_SHARED((NP, FH), jnp.float32),
            pltpu.VMEM_SHARED((NP, DEGW), jnp.float32),
            [pltpu.SemaphoreType.DMA for _ in range(NBUF)],
        ],
    )
    def _sc_agg(u_hbm, src_hbm, dst_hbm, zf_hbm, zd_hbm, ones_hbm,
                out_hbm, deg_hbm,
                src_v, dst_v, rows, ones_v, acc_sh, dacc_sh, sems):
        cid = lax.axis_index("c")
        sid = lax.axis_index("s")
        on0 = cid == 0
        uh = u_hbm.at[cid]

        pltpu.sync_copy(src_hbm.at[sid], src_v)
        pltpu.sync_copy(dst_hbm.at[sid], dst_v)
        if with_deg:
            pltpu.sync_copy(ones_hbm, ones_v)
        pltpu.sync_copy(zf_hbm, acc_sh.at[pl.ds(sid * STR, STR)])
        if with_deg:
            pltpu.sync_copy(zd_hbm, dacc_sh.at[pl.ds(sid * STR, STR)])
        plsc.subcore_barrier()

        def gather(j, b):
            return pltpu.make_async_copy(uh.at[src_v.at[j]], rows[b], sems[b])

        def scat(j, b):
            pltpu.sync_copy(rows[b], acc_sh.at[dst_v.at[j]], add=True)
            if with_deg:

                @pl.when(on0)
                def _():
                    pltpu.sync_copy(ones_v, dacc_sh.at[dst_v.at[j]], add=True)

        for b in range(NBUF):
            gather(b, b).start()

        def body(i, carry):
            j = NBUF * i
            for b in range(NBUF):
                gather(j + b, b).wait()
                scat(j + b, b)
                gather(j + b + NBUF, b).start()
            return carry

        lax.fori_loop(0, NCHUNK // NBUF - 1, body, 0)
        j_last = NCHUNK - NBUF
        for b in range(NBUF):
            gather(j_last + b, b).wait()
            scat(j_last + b, b)

        plsc.subcore_barrier()
        row = pl.ds(sid * STR, STR)
        pltpu.sync_copy(acc_sh.at[row], out_hbm.at[cid, row])
        if with_deg:
            pltpu.sync_copy(dacc_sh.at[row], deg_hbm.at[row])

    return _sc_agg


def kernel(x, edge_index, W1, b1, W2, b2, W3, b3, W4, b4, W5, b5, W6, b6):
    W1p = jnp.concatenate([W1, jnp.zeros((1, F), W1.dtype)], axis=0)
    Wst = jnp.stack([W1p, W2, W3, W4])
    bst = jnp.stack([b1, b2, b3, b4])[:, None, :]

    src = edge_index[0].reshape(NS, NCHUNK, CH)
    dst = edge_index[1].reshape(NS, NCHUNK, CH)
    zf = jnp.zeros((STR, FH), jnp.float32)
    zd = jnp.zeros((STR, DEGW), jnp.float32)
    ones = jnp.ones((CH, DEGW), jnp.float32)

    u4 = _tcA(x, Wst, bst)
    agg1, deg1 = _make_sc_agg(True)(u4, src, dst, zf, zd, ones)
    u5 = _tc_post(_tcB_body, agg1, deg1, W5, b5[None], final=False)
    agg2, _ = _make_sc_agg(False)(u5, src, dst, zf, zd, ones)
    return _tc_post(_tcC_body, agg2, deg1, W6, b6[None], final=True)


# speedup vs baseline: 12.5039x; 1.0580x over previous
import functools
import math

import jax
import jax.numpy as jnp
from jax import lax
from jax.experimental import pallas as pl
from jax.experimental.pallas import tpu as pltpu
from jax.experimental.pallas import tpu_sc as plsc

N = 10000
NH = N // 2
F = 128
E = 320000
MAXN = 1.0 - 1e-5

NC = 2
NS = 16
FH = F // NC
EPT = E // NS
CH = 125
NCHUNK = EPT // CH
NP = 10240
STR = NP // NS
DEGW = 8

BLKH = 1000
GRID = NH // BLKH


def _norm(p):
    return jnp.sqrt(jnp.sum(p * p, axis=-1, keepdims=True))


def _project(p):
    n = _norm(p)
    return jnp.where(n > MAXN, p / jnp.maximum(n, 1e-10) * MAXN, p)


def _expmap0(u):
    n = jnp.maximum(_norm(u), 1e-10)
    return _project(jnp.tanh(n) * u / n)


def _logmap0(p):
    p = _project(p)
    n = jnp.maximum(_norm(p), 1e-10)
    nc = jnp.minimum(n, MAXN)
    atanh = 0.5 * jnp.log((1.0 + nc) / (1.0 - nc))
    return atanh * p / n


TCUT = math.atanh(MAXN)


def _roundtrip(v):
    n = jnp.maximum(_norm(v), 1e-10)
    return jnp.where(n >= TCUT, TCUT / n, 1.0) * v


def _pack_store(o_ref, ut, ub):
    o_ref[0] = jnp.concatenate([ut[:, :FH], ub[:, :FH]], axis=1)
    o_ref[1] = jnp.concatenate([ut[:, FH:], ub[:, FH:]], axis=1)


def _poincare_tangent(x):
    x0 = x[:, 0:1]
    xs = jnp.concatenate([x[:, 1:], jnp.zeros_like(x0)], axis=1)
    return _logmap0(xs / (1.0 + x0))


def _tcA_body(xt_ref, xb_ref, W_ref, b_ref, o_ref):
    us = []
    for x in (xt_ref[...], xb_ref[...]):
        t = _poincare_tangent(x)
        for i in range(3):
            u = jnp.dot(t, W_ref[i],
                        preferred_element_type=jnp.float32) + b_ref[i]
            t = _roundtrip(jnp.maximum(u, 0.0))
        us.append(jnp.dot(t, W_ref[3],
                          preferred_element_type=jnp.float32) + b_ref[3])
    _pack_store(o_ref, us[0], us[1])


def _mean_relu_tangent(agg_ref, deg_ref):
    a0 = agg_ref[0]
    a1 = agg_ref[1]
    ts = []
    for sl, half in ((0, slice(0, FH)), (1, slice(FH, F))):
        s = jnp.concatenate([a0[:, half], a1[:, half]], axis=1)
        deg = jnp.sum(deg_ref[:, sl, :], axis=-1,
                      keepdims=True) * (1.0 / DEGW)
        u = s / jnp.maximum(deg, 1.0)
        ts.append(_roundtrip(jnp.maximum(u, 0.0)))
    return ts


def _tcB_body(agg_ref, deg_ref, W_ref, b_ref, o_ref):
    tt, tb = _mean_relu_tangent(agg_ref, deg_ref)
    ut = jnp.dot(tt, W_ref[...], preferred_element_type=jnp.float32) + b_ref[...]
    ub = jnp.dot(tb, W_ref[...], preferred_element_type=jnp.float32) + b_ref[...]
    _pack_store(o_ref, ut, ub)


def _tcC_body(agg_ref, deg_ref, W_ref, b_ref, o_ref):
    tt, tb = _mean_relu_tangent(agg_ref, deg_ref)
    ut = jnp.dot(tt, W_ref[...], preferred_element_type=jnp.float32) + b_ref[...]
    ub = jnp.dot(tb, W_ref[...], preferred_element_type=jnp.float32) + b_ref[...]
    o_ref[0] = _expmap0(ut)
    o_ref[1] = _expmap0(ub)


def _tcA(x, Wst, bst):
    return pl.pallas_call(
        _tcA_body,
        grid=(GRID,),
        in_specs=[
            pl.BlockSpec((BLKH, F), lambda i: (i, 0)),
            pl.BlockSpec((BLKH, F), lambda i: (i + GRID, 0)),
            pl.BlockSpec((4, F, F), lambda i: (0, 0, 0)),
            pl.BlockSpec((4, 1, F), lambda i: (0, 0, 0)),
        ],
        out_specs=pl.BlockSpec((NC, BLKH, F), lambda i: (0, i, 0)),
        out_shape=jax.ShapeDtypeStruct((NC, NH, F), jnp.float32),
    )(x, x, Wst, bst)


def _tc_post(body, agg, deg, W, b):
    return pl.pallas_call(
        body,
        grid=(GRID,),
        in_specs=[
            pl.BlockSpec((NC, BLKH, F), lambda i: (0, i, 0)),
            pl.BlockSpec((BLKH, NC, DEGW), lambda i: (i, 0, 0)),
            pl.BlockSpec((F, F), lambda i: (0, 0)),
            pl.BlockSpec((1, F), lambda i: (0, 0)),
        ],
        out_specs=pl.BlockSpec((NC, BLKH, F), lambda i: (0, i, 0)),
        out_shape=jax.ShapeDtypeStruct((NC, NH, F), jnp.float32),
    )(agg, deg, W, b)


NBUF = 4


@functools.cache
def _make_sc_agg(with_deg):
    mesh = plsc.VectorSubcoreMesh(
        core_axis_name="c", subcore_axis_name="s",
        num_cores=NC, num_subcores=NS)

    @functools.partial(
        pl.kernel,
        out_type=(
            jax.ShapeDtypeStruct((NC, NP, FH), jnp.float32),
            jax.ShapeDtypeStruct((NP, DEGW), jnp.float32),
        ),
        mesh=mesh,
        compiler_params=pltpu.CompilerParams(use_tc_tiling_on_sc=False),
        scratch_types=[
            pltpu.VMEM((NCHUNK, CH), jnp.int32),
            pltpu.VMEM((NCHUNK, CH), jnp.int32),
            [pltpu.VMEM((CH, FH), jnp.float32) for _ in range(NBUF)],
            pltpu.VMEM((CH, DEGW), jnp.float32),
            pltpu.VMEM_SHARED((NP, FH), jnp.float32),
            pltpu.VMEM_SHARED((NP, DEGW), jnp.float32),
            [pltpu.SemaphoreType.DMA for _ in range(NBUF)],
        ],
    )
    def _sc_agg(u_hbm, src_hbm, dst_hbm, zf_hbm, zd_hbm, ones_hbm,
                out_hbm, deg_hbm,
                src_v, dst_v, rows, ones_v, acc_sh, dacc_sh, sems):
        cid = lax.axis_index("c")
        sid = lax.axis_index("s")
        on0 = cid == 0
        uh = u_hbm.at[cid]

        pltpu.sync_copy(src_hbm.at[sid], src_v)
        pltpu.sync_copy(dst_hbm.at[sid], dst_v)
        if with_deg:
            pltpu.sync_copy(ones_hbm, ones_v)
        pltpu.sync_copy(zf_hbm, acc_sh.at[pl.ds(sid * STR, STR)])
        if with_deg:
            pltpu.sync_copy(zd_hbm, dacc_sh.at[pl.ds(sid * STR, STR)])
        plsc.subcore_barrier()

        def gather(j, b):
            return pltpu.make_async_copy(uh.at[src_v.at[j]], rows[b], sems[b])

        def scat(j, b):
            pltpu.sync_copy(rows[b], acc_sh.at[dst_v.at[j]], add=True)
            if with_deg:

                @pl.when(on0)
                def _():
                    pltpu.sync_copy(ones_v, dacc_sh.at[dst_v.at[j]], add=True)

        for b in range(NBUF):
            gather(b, b).start()

        def body(i, carry):
            j = NBUF * i
            for b in range(NBUF):
                gather(j + b, b).wait()
                scat(j + b, b)
                gather(j + b + NBUF, b).start()
            return carry

        lax.fori_loop(0, NCHUNK // NBUF - 1, body, 0)
        j_last = NCHUNK - NBUF
        for b in range(NBUF):
            gather(j_last + b, b).wait()
            scat(j_last + b, b)

        plsc.subcore_barrier()
        row = pl.ds(sid * STR, STR)
        pltpu.sync_copy(acc_sh.at[row], out_hbm.at[cid, row])
        if with_deg:
            pltpu.sync_copy(dacc_sh.at[row], deg_hbm.at[row])

    return _sc_agg


def kernel(x, edge_index, W1, b1, W2, b2, W3, b3, W4, b4, W5, b5, W6, b6):
    W1p = jnp.concatenate([W1, jnp.zeros((1, F), W1.dtype)], axis=0)
    Wst = jnp.stack([W1p, W2, W3, W4])
    bst = jnp.stack([b1, b2, b3, b4])[:, None, :]

    def perm(n):
        return jnp.where(n < NH, 2 * n, 2 * (n - NH) + 1)

    src = perm(edge_index[0]).reshape(NS, NCHUNK, CH)
    dst = perm(edge_index[1]).reshape(NS, NCHUNK, CH)
    zf = jnp.zeros((STR, FH), jnp.float32)
    zd = jnp.zeros((STR, DEGW), jnp.float32)
    ones = jnp.ones((CH, DEGW), jnp.float32)

    u4 = _tcA(x, Wst, bst).reshape(NC, N, FH)
    agg1, deg1 = _make_sc_agg(True)(u4, src, dst, zf, zd, ones)
    agg1r = agg1.reshape(NC, NP // 2, F)
    deg1r = deg1.reshape(NP // 2, NC, DEGW)
    u5 = _tc_post(_tcB_body, agg1r, deg1r, W5, b5[None]).reshape(NC, N, FH)
    agg2, _ = _make_sc_agg(False)(u5, src, dst, zf, zd, ones)
    agg2r = agg2.reshape(NC, NP // 2, F)
    out = _tc_post(_tcC_body, agg2r, deg1r, W6, b6[None])
    return out.reshape(N, F)
